# baseline (speedup 1.0000x reference)
import functools

import jax
import jax.numpy as jnp
from jax import lax
from jax.experimental import pallas as pl
from jax.experimental.pallas import tpu as pltpu
from jax.experimental.pallas import tpu_sc as plsc

NC = 2
NS = 16
NW = NC * NS
L = 16


def _sc_dist_sq(fpair, a_idx, p_idx, n_idx, N, B, C):
    D = fpair.shape[1] // 2
    T = a_idx.shape[0]
    TW = T // NW
    NCHUNK = TW // C
    NL = 3 * B

    mesh = plsc.VectorSubcoreMesh(
        core_axis_name="c", subcore_axis_name="s",
        num_cores=NC, num_subcores=NS)

    @functools.partial(
        pl.kernel,
        out_type=[jax.ShapeDtypeStruct((T,), jnp.float32),
                  jax.ShapeDtypeStruct((T,), jnp.float32)],
        mesh=mesh,
        scratch_types=(
            [pltpu.VMEM((TW,), jnp.int32) for _ in range(3)]
            + [pltpu.VMEM((TW,), jnp.int32) for _ in range(NL)]
            + [pltpu.VMEM((TW,), jnp.int32) for _ in range(3)]
            + [pltpu.VMEM((C, 2 * D), jnp.float32) for _ in range(NL)]
            + [pltpu.VMEM((TW,), jnp.float32),
               pltpu.VMEM((TW,), jnp.float32),
               pltpu.SemaphoreType.DMA]
        ),
        compiler_params=pltpu.CompilerParams(needs_layout_passes=False),
    )
    def k(f_hbm, ai_hbm, pi_hbm, ni_hbm, oap_hbm, oan_hbm, *sc):
        idx_v = sc[0:3]
        rix_v = sc[3:3 + NL]
        par_v = sc[3 + NL:6 + NL]
        rows_v = sc[6 + NL:6 + 2 * NL]
        oap_v, oan_v, sem = sc[6 + 2 * NL:]
        wid = lax.axis_index("s") * NC + lax.axis_index("c")
        base = wid * TW

        cps = [pltpu.async_copy(h.at[pl.ds(base, TW)], idx_v[i], sem)
               for i, h in enumerate((ai_hbm, pi_hbm, ni_hbm))]
        for cp in cps:
            cp.wait()

        def build(g, _):
            sl = pl.ds(g * L, L)
            for l in range(3):
                v = idx_v[l][sl]
                par_v[l][sl] = (v & 1) * D
                for b in range(B):
                    rix_v[l * B + b][sl] = (v + b * N) >> 1
            return 0

        lax.fori_loop(0, TW // L, build, 0)

        for ci in range(NCHUNK):
            cps = [pltpu.async_copy(
                       f_hbm.at[rix_v[lb].at[pl.ds(ci * C, C)]],
                       rows_v[lb], sem)
                   for lb in range(NL)]
            for cp in cps:
                cp.wait()

            def group(g, _):
                sl = pl.ds(ci * C + g * L, L)
                rows = g * L + lax.iota(jnp.int32, L)
                ca = par_v[0][sl]
                cp_ = par_v[1][sl]
                cn = par_v[2][sl]

                def dpos(d, accs):
                    acc_ap, acc_an = accs
                    ia = ca + d
                    ip = cp_ + d
                    in_ = cn + d
                    for b in range(B):
                        va = plsc.load_gather(rows_v[b], [rows, ia])
                        vp = plsc.load_gather(rows_v[B + b], [rows, ip])
                        vn = plsc.load_gather(rows_v[2 * B + b], [rows, in_])
                        dap = va - vp
                        dan = va - vn
                        acc_ap = acc_ap + dap * dap
                        acc_an = acc_an + dan * dan
                    return acc_ap, acc_an

                z = jnp.zeros((L,), jnp.float32)
                acc_ap, acc_an = lax.fori_loop(0, D, dpos, (z, z))
                oap_v[sl] = acc_ap
                oan_v[sl] = acc_an
                return 0

            lax.fori_loop(0, C // L, group, 0)

        pltpu.sync_copy(oap_v, oap_hbm.at[pl.ds(base, TW)])
        pltpu.sync_copy(oan_v, oan_hbm.at[pl.ds(base, TW)])

    return k(fpair, a_idx, p_idx, n_idx)


def _tc_finish(ap2, an2):
    def body(ap_ref, an_ref, o_ref):
        d = jnp.sqrt(ap_ref[...]) - jnp.sqrt(an_ref[...])
        o_ref[0, 0] = jnp.sum(jnp.maximum(d, 0.0))

    out = pl.pallas_call(
        body,
        out_shape=jax.ShapeDtypeStruct((1, 1), jnp.float32),
        out_specs=pl.BlockSpec(memory_space=pltpu.SMEM),
    )(ap2, an2)
    return out[0, 0]


def kernel(inr_features, anchor_idx, pos_idx, neg_idx):
    B, N, D = inr_features.shape
    T = anchor_idx.shape[0]
    fpair = inr_features.reshape(B * N // 2, 2 * D)
    ai = anchor_idx.astype(jnp.int32)
    pi = pos_idx.astype(jnp.int32)
    ni = neg_idx.astype(jnp.int32)
    ap2, an2 = _sc_dist_sq(fpair, ai, pi, ni, N, B, C=64)
    return _tc_finish(ap2.reshape(T // 128, 128), an2.reshape(T // 128, 128))


# speedup vs baseline: 1.5573x; 1.5573x over previous
import functools

import jax
import jax.numpy as jnp
from jax import lax
from jax.experimental import pallas as pl
from jax.experimental.pallas import tpu as pltpu
from jax.experimental.pallas import tpu_sc as plsc

NC = 2
NS = 16
NW = NC * NS
L = 16


def _tc_table(features, bn=2048):
    B, N, D = features.shape

    def body(in_ref, out_ref):
        for b in range(B):
            out_ref[:, b * D:(b + 1) * D] = in_ref[b]

    return pl.pallas_call(
        body,
        grid=(N // bn,),
        in_specs=[pl.BlockSpec((B, bn, D), lambda i: (0, i, 0))],
        out_specs=pl.BlockSpec((bn, B * D), lambda i: (i, 0)),
        out_shape=jax.ShapeDtypeStruct((N, B * D), jnp.float32),
    )(features)


def _sc_partial_sumsq(table, a_idx, p_idx, n_idx, C):
    W = table.shape[1]
    T = a_idx.shape[0]
    TW = T // NW
    NCHUNK = TW // C
    OR = TW // 8

    mesh = plsc.VectorSubcoreMesh(
        core_axis_name="c", subcore_axis_name="s",
        num_cores=NC, num_subcores=NS)

    @functools.partial(
        pl.kernel,
        out_type=[jax.ShapeDtypeStruct((T // 8, 128), jnp.float32),
                  jax.ShapeDtypeStruct((T // 8, 128), jnp.float32)],
        mesh=mesh,
        scratch_types=(
            [pltpu.VMEM((TW,), jnp.int32) for _ in range(3)]
            + [pltpu.VMEM((C, W), jnp.float32) for _ in range(3)]
            + [pltpu.VMEM((OR, 128), jnp.float32),
               pltpu.VMEM((OR, 128), jnp.float32),
               pltpu.SemaphoreType.DMA]
        ),
    )
    def k(tab_hbm, ai_hbm, pi_hbm, ni_hbm, oap_hbm, oan_hbm,
          ai_v, pi_v, ni_v, ra_v, rp_v, rn_v, oap_v, oan_v, sem):
        wid = lax.axis_index("s") * NC + lax.axis_index("c")
        base = wid * TW

        cps = [pltpu.async_copy(h.at[pl.ds(base, TW)], v, sem)
               for h, v in ((ai_hbm, ai_v), (pi_hbm, pi_v), (ni_hbm, ni_v))]
        for cp in cps:
            cp.wait()

        for ci in range(NCHUNK):
            cps = [pltpu.async_copy(tab_hbm.at[v.at[pl.ds(ci * C, C)]],
                                    r, sem)
                   for v, r in ((ai_v, ra_v), (pi_v, rp_v), (ni_v, rn_v))]
            for cp in cps:
                cp.wait()

            def trip(t, _):
                acc_ap = jnp.zeros((L,), jnp.float32)
                acc_an = jnp.zeros((L,), jnp.float32)
                for cc in range(W // L):
                    sl = pl.ds(cc * L, L)
                    va = ra_v[t, sl]
                    vp = rp_v[t, sl]
                    vn = rn_v[t, sl]
                    dap = va - vp
                    dan = va - vn
                    acc_ap = acc_ap + dap * dap
                    acc_an = acc_an + dan * dan
                tt = ci * C + t
                osl = pl.ds((tt & 7) * L, L)
                oap_v[tt >> 3, osl] = acc_ap
                oan_v[tt >> 3, osl] = acc_an
                return 0

            lax.fori_loop(0, C, trip, 0)

        pltpu.sync_copy(oap_v, oap_hbm.at[pl.ds(wid * OR, OR)])
        pltpu.sync_copy(oan_v, oan_hbm.at[pl.ds(wid * OR, OR)])

    return k(table, a_idx, p_idx, n_idx)


def _tc_finish(ap2, an2):
    def body(ap_ref, an_ref, o_ref):
        j = lax.broadcasted_iota(jnp.int32, (128, 8), 0)
        g = lax.broadcasted_iota(jnp.int32, (128, 8), 1)
        sel = (j // 16 == g).astype(jnp.float32)
        d2_ap = jnp.dot(ap_ref[...], sel, preferred_element_type=jnp.float32)
        d2_an = jnp.dot(an_ref[...], sel, preferred_element_type=jnp.float32)
        d = jnp.sqrt(d2_ap) - jnp.sqrt(d2_an)
        o_ref[0, 0] = jnp.sum(jnp.maximum(d, 0.0))

    out = pl.pallas_call(
        body,
        out_shape=jax.ShapeDtypeStruct((1, 1), jnp.float32),
        out_specs=pl.BlockSpec(memory_space=pltpu.SMEM),
    )(ap2, an2)
    return out[0, 0]


def kernel(inr_features, anchor_idx, pos_idx, neg_idx):
    table = _tc_table(inr_features)
    ai = anchor_idx.astype(jnp.int32)
    pi = pos_idx.astype(jnp.int32)
    ni = neg_idx.astype(jnp.int32)
    ap2, an2 = _sc_partial_sumsq(table, ai, pi, ni, C=64)
    return _tc_finish(ap2, an2)
